# baseline (speedup 1.0000x reference)
import dataclasses
import functools

import jax
import jax.numpy as jnp
from jax import lax
from jax.experimental import pallas as pl
from jax.experimental.pallas import tpu as pltpu
from jax.experimental.pallas import tpu_sc as plsc

_SC_PARAMS = pltpu.CompilerParams()
if "needs_layout_passes" in pltpu.CompilerParams.__dataclass_fields__:
    _SC_PARAMS = dataclasses.replace(_SC_PARAMS, needs_layout_passes=False)

_K = 10
_BLK = 1024
_C = 128

_GDN = lax.GatherDimensionNumbers(
    offset_dims=(), collapsed_slice_dims=(0,), start_index_map=(0,))


def _splat(v16, j):
    idx = jnp.full((16, 1), j, jnp.int32)
    return lax.gather(v16, idx, _GDN, (1,),
                      mode=lax.GatherScatterMode.PROMISE_IN_BOUNDS)


def _proj_body(c_ref, w_ref, z_ref):
    z_ref[...] = jnp.dot(c_ref[...], w_ref[...],
                         preferred_element_type=jnp.float32)


def _proj(c, w_all):
    npad, cin = c.shape
    kh = w_all.shape[1]
    return pl.pallas_call(
        _proj_body,
        grid=(npad // _BLK,),
        in_specs=[
            pl.BlockSpec((_BLK, cin), lambda i: (i, 0)),
            pl.BlockSpec((cin, kh), lambda i: (0, 0)),
        ],
        out_specs=pl.BlockSpec((_BLK, kh), lambda i: (i, 0)),
        out_shape=jax.ShapeDtypeStruct((npad, kh), jnp.float32),
    )(c, w_all)


def _add2_body(p_ref, y_ref):
    y_ref[...] = p_ref[0] + p_ref[1]


def _add2(p):
    _, npad, hid = p.shape
    return pl.pallas_call(
        _add2_body,
        grid=(npad // _BLK,),
        in_specs=[pl.BlockSpec((2, _BLK, hid), lambda i: (0, i, 0))],
        out_specs=pl.BlockSpec((_BLK, hid), lambda i: (i, 0)),
        out_shape=jax.ShapeDtypeStruct((npad, hid), jnp.float32),
    )(p)


def _gate_a_body(g1_ref, h_ref, b_ref, u_ref, rh_ref):
    u = jax.nn.sigmoid(g1_ref[...] + b_ref[...])
    u_ref[...] = u
    rh_ref[...] = u * h_ref[...]


def _gate_a(g1, h, b2d):
    npad, hid = g1.shape
    return pl.pallas_call(
        _gate_a_body,
        grid=(npad // _BLK,),
        in_specs=[
            pl.BlockSpec((_BLK, hid), lambda i: (i, 0)),
            pl.BlockSpec((_BLK, hid), lambda i: (i, 0)),
            pl.BlockSpec((1, hid), lambda i: (0, 0)),
        ],
        out_specs=[
            pl.BlockSpec((_BLK, hid), lambda i: (i, 0)),
            pl.BlockSpec((_BLK, hid), lambda i: (i, 0)),
        ],
        out_shape=[
            jax.ShapeDtypeStruct((npad, hid), jnp.float32),
            jax.ShapeDtypeStruct((npad, hid), jnp.float32),
        ],
    )(g1, h, b2d)


def _gate_b_body(g2_ref, u_ref, rh_ref, b_ref, h_ref):
    cc = jax.nn.sigmoid(g2_ref[...] + b_ref[...])
    h_ref[...] = rh_ref[...] + cc - u_ref[...] * cc


def _gate_b(g2, u, rh, b2d):
    npad, hid = g2.shape
    return pl.pallas_call(
        _gate_b_body,
        grid=(npad // _BLK,),
        in_specs=[
            pl.BlockSpec((_BLK, hid), lambda i: (i, 0)),
            pl.BlockSpec((_BLK, hid), lambda i: (i, 0)),
            pl.BlockSpec((_BLK, hid), lambda i: (i, 0)),
            pl.BlockSpec((1, hid), lambda i: (0, 0)),
        ],
        out_specs=pl.BlockSpec((_BLK, hid), lambda i: (i, 0)),
        out_shape=jax.ShapeDtypeStruct((npad, hid), jnp.float32),
    )(g2, u, rh, b2d)


@functools.lru_cache(maxsize=None)
def _make_spmv(npad, n, hid):
    mesh = plsc.VectorSubcoreMesh(core_axis_name="c", subcore_axis_name="s")
    nhalf = npad // 2
    arows = npad // 32

    @functools.partial(
        pl.kernel,
        out_type=jax.ShapeDtypeStruct((npad, hid), jnp.float32),
        mesh=mesh,
        compiler_params=_SC_PARAMS,
        scratch_types=[
            pltpu.VMEM_SHARED((nhalf, hid), jnp.float32),
            pltpu.VMEM((64,), jnp.int32),
            pltpu.VMEM((3, _C), jnp.int32),
            pltpu.VMEM((3, _C), jnp.int32),
            pltpu.VMEM((_C, hid), jnp.float32),
            pltpu.VMEM((_C, hid), jnp.float32),
            pltpu.SemaphoreType.DMA,
            pltpu.SemaphoreType.DMA,
        ],
    )
    def spmv(y_hbm, z_hbm, comb_hbm, cb_hbm, out_hbm,
             acc_sh, cb_v, idx_a, idx_b, rows_a, rows_b, sem_a, sem_b):
        c = lax.axis_index("c")
        s = lax.axis_index("s")
        wid = c * 16 + s

        pltpu.sync_copy(z_hbm.at[pl.ds(wid * arows, arows)],
                        acc_sh.at[pl.ds(s * arows, arows)])
        pltpu.sync_copy(cb_hbm, cb_v)

        def lane(x):
            off = (x // 16) * 16
            v = cb_v[pl.ds(off, 16)]
            return jnp.max(jnp.where(lax.iota(jnp.int32, 16) == x - off,
                                     v, 0))

        start = lane(wid)
        end = lane(wid + 1)

        def scale_scatter(idx_r, rows_r):
            @pl.loop(0, _C // 16)
            def _(g):
                wv = plsc.bitcast(idx_r[2, pl.ds(g * 16, 16)], jnp.float32)
                for j in range(16):
                    sp = _splat(wv, j)
                    ej = g * 16 + j
                    for q in range(hid // 16):
                        sl = (ej, pl.ds(q * 16, 16))
                        rows_r[sl] = rows_r[sl] * sp

            pltpu.sync_copy(rows_r, acc_sh.at[idx_r.at[1]], add=True)

        pltpu.sync_copy(comb_hbm.at[start], idx_a)
        pltpu.async_copy(y_hbm.at[idx_a.at[0]], rows_a, sem_a)

        def body(i, carry):
            qb = start + 2 * i + 1
            pltpu.sync_copy(comb_hbm.at[qb], idx_b)
            pltpu.async_copy(y_hbm.at[idx_b.at[0]], rows_b, sem_b)
            pltpu.make_async_copy(y_hbm.at[idx_a.at[0]], rows_a, sem_a).wait()
            scale_scatter(idx_a, rows_a)
            pltpu.sync_copy(comb_hbm.at[qb + 1], idx_a)
            pltpu.async_copy(y_hbm.at[idx_a.at[0]], rows_a, sem_a)
            pltpu.make_async_copy(y_hbm.at[idx_b.at[0]], rows_b, sem_b).wait()
            scale_scatter(idx_b, rows_b)
            return carry

        lax.fori_loop(0, (end - start) // 2, body, 0)
        pltpu.make_async_copy(y_hbm.at[idx_a.at[0]], rows_a, sem_a).wait()
        pltpu.sync_copy(acc_sh.at[pl.ds(s * arows, arows)],
                        out_hbm.at[pl.ds(wid * arows, arows)])

    return spmv


def _gconv(c, w_all, edges, npad, n):
    hid = c.shape[1] // 2
    z = _proj(c, w_all)
    spmv = _make_spmv(npad, n, hid)
    comb, cb = edges
    y = z[:, _K * hid:(_K + 1) * hid]
    for k in range(_K - 1, -1, -1):
        zk = z[:, k * hid:(k + 1) * hid]
        y = spmv(y, zk, comb, cb)
    return y


def kernel(input, hidden, edge_index, edge_weight, W, b):
    seq, n, in_dim = input.shape
    hid = hidden.shape[2]
    cin = in_dim + hid
    e = edge_index.shape[1]
    npad = ((n + _BLK - 1) // _BLK) * _BLK

    arows = npad // 32
    nhalf = npad // 2
    order = jnp.argsort(edge_index[1])
    srcs = edge_index[0][order]
    dsts = edge_index[1][order]
    ws = edge_weight[order]
    tile_of = dsts // arows
    seg_start = jnp.searchsorted(
        dsts, jnp.arange(33, dtype=jnp.int32) * arows).astype(jnp.int32)
    cnt_t = seg_start[1:] - seg_start[:-1]
    aligned = ((cnt_t + 2 * _C - 1) // (2 * _C)) * (2 * _C)
    astart = jnp.concatenate(
        [jnp.zeros((1,), jnp.int32), jnp.cumsum(aligned).astype(jnp.int32)])
    ep = (-(-e // (2 * _C)) + 33) * (2 * _C)
    newpos = (jnp.arange(e, dtype=jnp.int32) - seg_start[tile_of]
              + astart[tile_of])
    pos = jnp.arange(ep, dtype=jnp.int32)
    pos_tile = jnp.clip(
        jnp.searchsorted(astart, pos, side="right").astype(jnp.int32) - 1,
        0, 31)
    src_p = jnp.zeros((ep,), jnp.int32).at[newpos].set(srcs)
    ldst_p = ((pos_tile % 16) * arows).at[newpos].set(dsts % nhalf)
    w_p = jnp.zeros((ep,), jnp.float32).at[newpos].set(ws)
    comb = jnp.stack(
        [src_p.reshape(-1, _C), ldst_p.reshape(-1, _C),
         w_p.view(jnp.int32).reshape(-1, _C)], axis=1)
    cb = jnp.zeros((64,), jnp.int32).at[:33].set(astart // _C)
    edges = (comb, cb)

    w_all = W.reshape(_K + 1, cin, hid).transpose(1, 0, 2).reshape(
        cin, (_K + 1) * hid)
    b2d = b[None, :]

    h = jnp.zeros((npad, hid), jnp.float32).at[:n].set(hidden[0])
    xpad = jnp.zeros((seq, npad, in_dim), jnp.float32).at[:, :n].set(input)

    outs = []
    for t in range(seq):
        x = xpad[t]
        c1 = jnp.concatenate([x, h], axis=1)
        g1 = _gconv(c1, w_all, edges, npad, n)
        u, rh = _gate_a(g1, h, b2d)
        c2 = jnp.concatenate([x, rh], axis=1)
        g2 = _gconv(c2, w_all, edges, npad, n)
        h = _gate_b(g2, u, rh, b2d)
        outs.append(h[:n])

    output = jnp.stack(outs, axis=0)
    return (output, output[seq - 1][None, :, :])


# speedup vs baseline: 1.7453x; 1.7453x over previous
import dataclasses
import functools

import jax
import jax.numpy as jnp
from jax import lax
from jax.experimental import pallas as pl
from jax.experimental.pallas import tpu as pltpu
from jax.experimental.pallas import tpu_sc as plsc

_SC_PARAMS = pltpu.CompilerParams()
if "needs_layout_passes" in pltpu.CompilerParams.__dataclass_fields__:
    _SC_PARAMS = dataclasses.replace(_SC_PARAMS, needs_layout_passes=False)

_K = 10
_BLK = 1024
_C = 64

_GDN = lax.GatherDimensionNumbers(
    offset_dims=(), collapsed_slice_dims=(0,), start_index_map=(0,))


def _splat(v16, j):
    idx = jnp.full((16, 1), j, jnp.int32)
    return lax.gather(v16, idx, _GDN, (1,),
                      mode=lax.GatherScatterMode.PROMISE_IN_BOUNDS)


def _proj_body(c_ref, w_ref, z_ref):
    z_ref[...] = jnp.dot(c_ref[...], w_ref[...],
                         preferred_element_type=jnp.float32)


def _proj(c, w_all):
    npad, cin = c.shape
    kh = w_all.shape[1]
    return pl.pallas_call(
        _proj_body,
        grid=(npad // _BLK,),
        in_specs=[
            pl.BlockSpec((_BLK, cin), lambda i: (i, 0)),
            pl.BlockSpec((cin, kh), lambda i: (0, 0)),
        ],
        out_specs=pl.BlockSpec((_BLK, kh), lambda i: (i, 0)),
        out_shape=jax.ShapeDtypeStruct((npad, kh), jnp.float32),
    )(c, w_all)


def _add2_body(p_ref, y_ref):
    y_ref[...] = p_ref[0] + p_ref[1]


def _add2(p):
    _, npad, hid = p.shape
    return pl.pallas_call(
        _add2_body,
        grid=(npad // _BLK,),
        in_specs=[pl.BlockSpec((2, _BLK, hid), lambda i: (0, i, 0))],
        out_specs=pl.BlockSpec((_BLK, hid), lambda i: (i, 0)),
        out_shape=jax.ShapeDtypeStruct((npad, hid), jnp.float32),
    )(p)


def _gate_a_body(g1_ref, h_ref, b_ref, u_ref, rh_ref):
    u = jax.nn.sigmoid(g1_ref[...] + b_ref[...])
    u_ref[...] = u
    rh_ref[...] = u * h_ref[...]


def _gate_a(g1, h, b2d):
    npad, hid = g1.shape
    return pl.pallas_call(
        _gate_a_body,
        grid=(npad // _BLK,),
        in_specs=[
            pl.BlockSpec((_BLK, hid), lambda i: (i, 0)),
            pl.BlockSpec((_BLK, hid), lambda i: (i, 0)),
            pl.BlockSpec((1, hid), lambda i: (0, 0)),
        ],
        out_specs=[
            pl.BlockSpec((_BLK, hid), lambda i: (i, 0)),
            pl.BlockSpec((_BLK, hid), lambda i: (i, 0)),
        ],
        out_shape=[
            jax.ShapeDtypeStruct((npad, hid), jnp.float32),
            jax.ShapeDtypeStruct((npad, hid), jnp.float32),
        ],
    )(g1, h, b2d)


def _gate_b_body(g2_ref, u_ref, rh_ref, b_ref, h_ref):
    cc = jax.nn.sigmoid(g2_ref[...] + b_ref[...])
    h_ref[...] = rh_ref[...] + cc - u_ref[...] * cc


def _gate_b(g2, u, rh, b2d):
    npad, hid = g2.shape
    return pl.pallas_call(
        _gate_b_body,
        grid=(npad // _BLK,),
        in_specs=[
            pl.BlockSpec((_BLK, hid), lambda i: (i, 0)),
            pl.BlockSpec((_BLK, hid), lambda i: (i, 0)),
            pl.BlockSpec((_BLK, hid), lambda i: (i, 0)),
            pl.BlockSpec((1, hid), lambda i: (0, 0)),
        ],
        out_specs=pl.BlockSpec((_BLK, hid), lambda i: (i, 0)),
        out_shape=jax.ShapeDtypeStruct((npad, hid), jnp.float32),
    )(g2, u, rh, b2d)


@functools.lru_cache(maxsize=None)
def _make_spmv(npad, n, hid):
    mesh = plsc.VectorSubcoreMesh(core_axis_name="c", subcore_axis_name="s")
    nhalf = npad // 2
    yrows = -(-n // (16 * 8)) * 8
    lastrows = n - 15 * yrows
    arows = npad // 32

    @functools.partial(
        pl.kernel,
        out_type=jax.ShapeDtypeStruct((npad, hid), jnp.float32),
        mesh=mesh,
        compiler_params=_SC_PARAMS,
        scratch_types=[
            pltpu.VMEM_SHARED((n, hid), jnp.float32),
            pltpu.VMEM_SHARED((nhalf, hid), jnp.float32),
            pltpu.VMEM((64,), jnp.int32),
            pltpu.VMEM((3, _C), jnp.int32),
            pltpu.VMEM((3, _C), jnp.int32),
            pltpu.VMEM((_C, hid), jnp.float32),
            pltpu.SemaphoreType.DMA,
            pltpu.SemaphoreType.DMA,
        ],
    )
    def spmv(y_hbm, z_hbm, comb_hbm, cb_hbm, out_hbm,
             y_sh, acc_sh, cb_v, idx_a, idx_b, rows_v, sem_g, sem_i):
        c = lax.axis_index("c")
        s = lax.axis_index("s")
        wid = c * 16 + s

        @pl.when(s < 15)
        def _():
            pltpu.sync_copy(y_hbm.at[pl.ds(s * yrows, yrows)],
                            y_sh.at[pl.ds(s * yrows, yrows)])

        @pl.when(s == 15)
        def _():
            pltpu.sync_copy(y_hbm.at[pl.ds(15 * yrows, lastrows)],
                            y_sh.at[pl.ds(15 * yrows, lastrows)])
        pltpu.sync_copy(z_hbm.at[pl.ds(wid * arows, arows)],
                        acc_sh.at[pl.ds(s * arows, arows)])
        pltpu.sync_copy(cb_hbm, cb_v)

        def lane(x):
            off = (x // 16) * 16
            v = cb_v[pl.ds(off, 16)]
            return jnp.max(jnp.where(lax.iota(jnp.int32, 16) == x - off,
                                     v, 0))

        start = lane(wid)
        end = lane(wid + 1)
        plsc.subcore_barrier()

        def process(idx_r, idx_next, q_next):
            pltpu.async_copy(comb_hbm.at[q_next], idx_next, sem_i)
            pltpu.async_copy(y_sh.at[idx_r.at[0]], rows_v, sem_g).wait()

            @pl.loop(0, _C // 16)
            def _(g):
                wv = plsc.bitcast(idx_r[2, pl.ds(g * 16, 16)], jnp.float32)
                for j in range(16):
                    sp = _splat(wv, j)
                    ej = g * 16 + j
                    for q in range(hid // 16):
                        sl = (ej, pl.ds(q * 16, 16))
                        rows_v[sl] = rows_v[sl] * sp

            pltpu.sync_copy(rows_v, acc_sh.at[idx_r.at[1]], add=True)
            pltpu.make_async_copy(comb_hbm.at[q_next], idx_next, sem_i).wait()

        pltpu.sync_copy(comb_hbm.at[start], idx_a)

        def body(i, carry):
            q = start + 2 * i
            process(idx_a, idx_b, q + 1)
            process(idx_b, idx_a, q + 2)
            return carry

        lax.fori_loop(0, (end - start) // 2, body, 0)
        pltpu.sync_copy(acc_sh.at[pl.ds(s * arows, arows)],
                        out_hbm.at[pl.ds(wid * arows, arows)])

    return spmv


def _gconv(c, w_all, edges, npad, n):
    hid = c.shape[1] // 2
    z = _proj(c, w_all)
    spmv = _make_spmv(npad, n, hid)
    comb, cb = edges
    y = z[:, _K * hid:(_K + 1) * hid]
    for k in range(_K - 1, -1, -1):
        zk = z[:, k * hid:(k + 1) * hid]
        y = spmv(y, zk, comb, cb)
    return y


def kernel(input, hidden, edge_index, edge_weight, W, b):
    seq, n, in_dim = input.shape
    hid = hidden.shape[2]
    cin = in_dim + hid
    e = edge_index.shape[1]
    npad = ((n + _BLK - 1) // _BLK) * _BLK

    arows = npad // 32
    nhalf = npad // 2
    order = jnp.argsort(edge_index[1])
    srcs = edge_index[0][order]
    dsts = edge_index[1][order]
    ws = edge_weight[order]
    tile_of = dsts // arows
    seg_start = jnp.searchsorted(
        dsts, jnp.arange(33, dtype=jnp.int32) * arows).astype(jnp.int32)
    cnt_t = seg_start[1:] - seg_start[:-1]
    aligned = ((cnt_t + 2 * _C - 1) // (2 * _C)) * (2 * _C)
    astart = jnp.concatenate(
        [jnp.zeros((1,), jnp.int32), jnp.cumsum(aligned).astype(jnp.int32)])
    ep = (-(-e // (2 * _C)) + 33) * (2 * _C)
    newpos = (jnp.arange(e, dtype=jnp.int32) - seg_start[tile_of]
              + astart[tile_of])
    pos = jnp.arange(ep, dtype=jnp.int32)
    pos_tile = jnp.clip(
        jnp.searchsorted(astart, pos, side="right").astype(jnp.int32) - 1,
        0, 31)
    src_p = jnp.zeros((ep,), jnp.int32).at[newpos].set(srcs)
    ldst_p = ((pos_tile % 16) * arows).at[newpos].set(dsts % nhalf)
    w_p = jnp.zeros((ep,), jnp.float32).at[newpos].set(ws)
    comb = jnp.stack(
        [src_p.reshape(-1, _C), ldst_p.reshape(-1, _C),
         w_p.view(jnp.int32).reshape(-1, _C)], axis=1)
    cb = jnp.zeros((64,), jnp.int32).at[:33].set(astart // _C)
    edges = (comb, cb)

    w_all = W.reshape(_K + 1, cin, hid).transpose(1, 0, 2).reshape(
        cin, (_K + 1) * hid)
    b2d = b[None, :]

    h = jnp.zeros((npad, hid), jnp.float32).at[:n].set(hidden[0])
    xpad = jnp.zeros((seq, npad, in_dim), jnp.float32).at[:, :n].set(input)

    outs = []
    for t in range(seq):
        x = xpad[t]
        c1 = jnp.concatenate([x, h], axis=1)
        g1 = _gconv(c1, w_all, edges, npad, n)
        u, rh = _gate_a(g1, h, b2d)
        c2 = jnp.concatenate([x, rh], axis=1)
        g2 = _gconv(c2, w_all, edges, npad, n)
        h = _gate_b(g2, u, rh, b2d)
        outs.append(h[:n])

    output = jnp.stack(outs, axis=0)
    return (output, output[seq - 1][None, :, :])


# speedup vs baseline: 1.7463x; 1.0005x over previous
import dataclasses
import functools

import jax
import jax.numpy as jnp
from jax import lax
from jax.experimental import pallas as pl
from jax.experimental.pallas import tpu as pltpu
from jax.experimental.pallas import tpu_sc as plsc

_SC_PARAMS = pltpu.CompilerParams()
if "needs_layout_passes" in pltpu.CompilerParams.__dataclass_fields__:
    _SC_PARAMS = dataclasses.replace(_SC_PARAMS, needs_layout_passes=False)

_K = 10
_BLK = 1024
_C = 64

_GDN = lax.GatherDimensionNumbers(
    offset_dims=(), collapsed_slice_dims=(0,), start_index_map=(0,))


def _splat(v16, j):
    idx = jnp.full((16, 1), j, jnp.int32)
    return lax.gather(v16, idx, _GDN, (1,),
                      mode=lax.GatherScatterMode.PROMISE_IN_BOUNDS)


def _proj_body(c_ref, w_ref, z_ref):
    z_ref[...] = jnp.dot(c_ref[...], w_ref[...],
                         preferred_element_type=jnp.float32)


def _proj(c, w_all):
    npad, cin = c.shape
    kh = w_all.shape[1]
    return pl.pallas_call(
        _proj_body,
        grid=(npad // _BLK,),
        in_specs=[
            pl.BlockSpec((_BLK, cin), lambda i: (i, 0)),
            pl.BlockSpec((cin, kh), lambda i: (0, 0)),
        ],
        out_specs=pl.BlockSpec((_BLK, kh), lambda i: (i, 0)),
        out_shape=jax.ShapeDtypeStruct((npad, kh), jnp.float32),
    )(c, w_all)


def _gate_a_body(g1_ref, h_ref, b_ref, u_ref, rh_ref):
    u = jax.nn.sigmoid(g1_ref[...] + b_ref[...])
    u_ref[...] = u
    rh_ref[...] = u * h_ref[...]


def _gate_a(g1, h, b2d):
    npad, hid = g1.shape
    return pl.pallas_call(
        _gate_a_body,
        grid=(npad // _BLK,),
        in_specs=[
            pl.BlockSpec((_BLK, hid), lambda i: (i, 0)),
            pl.BlockSpec((_BLK, hid), lambda i: (i, 0)),
            pl.BlockSpec((1, hid), lambda i: (0, 0)),
        ],
        out_specs=[
            pl.BlockSpec((_BLK, hid), lambda i: (i, 0)),
            pl.BlockSpec((_BLK, hid), lambda i: (i, 0)),
        ],
        out_shape=[
            jax.ShapeDtypeStruct((npad, hid), jnp.float32),
            jax.ShapeDtypeStruct((npad, hid), jnp.float32),
        ],
    )(g1, h, b2d)


def _gate_b_body(g2_ref, u_ref, rh_ref, b_ref, h_ref):
    cc = jax.nn.sigmoid(g2_ref[...] + b_ref[...])
    h_ref[...] = rh_ref[...] + cc - u_ref[...] * cc


def _gate_b(g2, u, rh, b2d):
    npad, hid = g2.shape
    return pl.pallas_call(
        _gate_b_body,
        grid=(npad // _BLK,),
        in_specs=[
            pl.BlockSpec((_BLK, hid), lambda i: (i, 0)),
            pl.BlockSpec((_BLK, hid), lambda i: (i, 0)),
            pl.BlockSpec((_BLK, hid), lambda i: (i, 0)),
            pl.BlockSpec((1, hid), lambda i: (0, 0)),
        ],
        out_specs=pl.BlockSpec((_BLK, hid), lambda i: (i, 0)),
        out_shape=jax.ShapeDtypeStruct((npad, hid), jnp.float32),
    )(g2, u, rh, b2d)


@functools.lru_cache(maxsize=None)
def _make_spmv(npad, n, hid):
    mesh = plsc.VectorSubcoreMesh(core_axis_name="c", subcore_axis_name="s")
    nhalf = npad // 2
    yrows = -(-n // (16 * 8)) * 8
    lastrows = n - 15 * yrows
    arows = npad // 32

    @functools.partial(
        pl.kernel,
        out_type=jax.ShapeDtypeStruct((npad, hid), jnp.float32),
        mesh=mesh,
        compiler_params=_SC_PARAMS,
        scratch_types=[
            pltpu.VMEM_SHARED((n, hid), jnp.float32),
            pltpu.VMEM_SHARED((nhalf, hid), jnp.float32),
            pltpu.VMEM((64,), jnp.int32),
            pltpu.VMEM((3, _C), jnp.int32),
            pltpu.VMEM((3, _C), jnp.int32),
            pltpu.VMEM((_C, hid), jnp.float32),
            pltpu.SemaphoreType.DMA,
            pltpu.SemaphoreType.DMA,
        ],
    )
    def spmv(y_hbm, z_hbm, comb_hbm, cb_hbm, out_hbm,
             y_sh, acc_sh, cb_v, idx_a, idx_b, rows_v, sem_g, sem_i):
        c = lax.axis_index("c")
        s = lax.axis_index("s")
        wid = c * 16 + s

        @pl.when(s < 15)
        def _():
            pltpu.sync_copy(y_hbm.at[pl.ds(s * yrows, yrows)],
                            y_sh.at[pl.ds(s * yrows, yrows)])

        @pl.when(s == 15)
        def _():
            pltpu.sync_copy(y_hbm.at[pl.ds(15 * yrows, lastrows)],
                            y_sh.at[pl.ds(15 * yrows, lastrows)])
        pltpu.sync_copy(z_hbm.at[pl.ds(wid * arows, arows)],
                        acc_sh.at[pl.ds(s * arows, arows)])
        pltpu.sync_copy(cb_hbm, cb_v)

        def lane(x):
            off = (x // 16) * 16
            v = cb_v[pl.ds(off, 16)]
            return jnp.max(jnp.where(lax.iota(jnp.int32, 16) == x - off,
                                     v, 0))

        start = lane(wid)
        end = lane(wid + 1)
        plsc.subcore_barrier()

        def process(idx_r, idx_next, q_next):
            pltpu.async_copy(comb_hbm.at[q_next], idx_next, sem_i)
            pltpu.async_copy(y_sh.at[idx_r.at[0]], rows_v, sem_g).wait()

            @pl.loop(0, _C // 16)
            def _(g):
                wv = plsc.bitcast(idx_r[2, pl.ds(g * 16, 16)], jnp.float32)
                for j in range(16):
                    sp = _splat(wv, j)
                    ej = g * 16 + j
                    for q in range(hid // 16):
                        sl = (ej, pl.ds(q * 16, 16))
                        rows_v[sl] = rows_v[sl] * sp

            pltpu.sync_copy(rows_v, acc_sh.at[idx_r.at[1]], add=True)
            pltpu.make_async_copy(comb_hbm.at[q_next], idx_next, sem_i).wait()

        pltpu.sync_copy(comb_hbm.at[start], idx_a)

        def body(i, carry):
            q = start + 2 * i
            process(idx_a, idx_b, q + 1)
            process(idx_b, idx_a, q + 2)
            return carry

        lax.fori_loop(0, (end - start) // 2, body, 0)
        pltpu.sync_copy(acc_sh.at[pl.ds(s * arows, arows)],
                        out_hbm.at[pl.ds(wid * arows, arows)])

    return spmv


def _gconv(c, w_all, edges, npad, n):
    hid = c.shape[1] // 2
    z = _proj(c, w_all)
    spmv = _make_spmv(npad, n, hid)
    comb, cb = edges
    y = z[:, _K * hid:(_K + 1) * hid]
    for k in range(_K - 1, -1, -1):
        zk = z[:, k * hid:(k + 1) * hid]
        y = spmv(y, zk, comb, cb)
    return y


def kernel(input, hidden, edge_index, edge_weight, W, b):
    seq, n, in_dim = input.shape
    hid = hidden.shape[2]
    cin = in_dim + hid
    e = edge_index.shape[1]
    npad = ((n + _BLK - 1) // _BLK) * _BLK

    arows = npad // 32
    nhalf = npad // 2
    order = jnp.argsort(edge_index[1])
    srcs = edge_index[0][order]
    dsts = edge_index[1][order]
    ws = edge_weight[order]
    tile_of = dsts // arows
    seg_start = jnp.searchsorted(
        dsts, jnp.arange(33, dtype=jnp.int32) * arows).astype(jnp.int32)
    cnt_t = seg_start[1:] - seg_start[:-1]
    aligned = ((cnt_t + 2 * _C - 1) // (2 * _C)) * (2 * _C)
    astart = jnp.concatenate(
        [jnp.zeros((1,), jnp.int32), jnp.cumsum(aligned).astype(jnp.int32)])
    ep = (-(-e // (2 * _C)) + 33) * (2 * _C)
    newpos = (jnp.arange(e, dtype=jnp.int32) - seg_start[tile_of]
              + astart[tile_of])
    pos = jnp.arange(ep, dtype=jnp.int32)
    pos_tile = jnp.clip(
        jnp.searchsorted(astart, pos, side="right").astype(jnp.int32) - 1,
        0, 31)
    src_p = jnp.zeros((ep,), jnp.int32).at[newpos].set(srcs)
    ldst_p = ((pos_tile % 16) * arows).at[newpos].set(dsts % nhalf)
    w_p = jnp.zeros((ep,), jnp.float32).at[newpos].set(ws)
    comb = jnp.stack(
        [src_p.reshape(-1, _C), ldst_p.reshape(-1, _C),
         w_p.view(jnp.int32).reshape(-1, _C)], axis=1)
    cb = jnp.zeros((64,), jnp.int32).at[:33].set(astart // _C)
    edges = (comb, cb)

    w_all = W.reshape(_K + 1, cin, hid).transpose(1, 0, 2).reshape(
        cin, (_K + 1) * hid)
    b2d = b[None, :]

    h = jnp.zeros((npad, hid), jnp.float32).at[:n].set(hidden[0])
    xpad = jnp.zeros((seq, npad, in_dim), jnp.float32).at[:, :n].set(input)

    outs = []
    for t in range(seq):
        x = xpad[t]
        c1 = jnp.concatenate([x, h], axis=1)
        g1 = _gconv(c1, w_all, edges, npad, n)
        u, rh = _gate_a(g1, h, b2d)
        c2 = jnp.concatenate([x, rh], axis=1)
        g2 = _gconv(c2, w_all, edges, npad, n)
        h = _gate_b(g2, u, rh, b2d)
        outs.append(h[:n])

    output = jnp.stack(outs, axis=0)
    return (output, output[seq - 1][None, :, :])
